# baseline (speedup 1.0000x reference)
import functools

import jax
import jax.numpy as jnp
from jax import lax
from jax.experimental import pallas as pl
from jax.experimental.pallas import tpu as pltpu


def _gat_exact(A, X, W, a_src, a_dst, b, heads):
    N = X.shape[0]
    xp = (X @ W).reshape(N, heads, -1)
    asrc = jnp.sum(xp * a_src[None, :, :], axis=-1)
    adst = jnp.sum(xp * a_dst[None, :, :], axis=-1)
    e = adst[:, None, :] + asrc[None, :, :]
    e = jnp.where(e >= 0, e, 0.2 * e)
    mask = (A != 0) | jnp.eye(N, dtype=bool)
    e = jnp.where(mask[:, :, None], e, -1e30)
    attn = jax.nn.softmax(e, axis=1)
    out = jnp.einsum('ijh,jhd->ihd', attn, xp).reshape(N, -1)
    return out + b


def _sym_norm_exact(A):
    d = A.sum(axis=1) + 1e-5
    dinv = d ** -0.5
    return dinv[:, None] * A * dinv[None, :]


def _prep_body(x_ref, w_ref, m_ref, xp_ref, aa_ref, aat_ref):
    xp = jnp.dot(x_ref[...], w_ref[...], preferred_element_type=jnp.float32)
    xp_ref[...] = xp
    aa = jnp.dot(xp, m_ref[...], preferred_element_type=jnp.float32)
    aa_ref[...] = aa
    aat_ref[...] = aa.T


def _prep(X, W, M):
    N = X.shape[0]
    B = min(1024, N)
    return pl.pallas_call(
        _prep_body,
        grid=(N // B,),
        in_specs=[
            pl.BlockSpec((B, 128), lambda i: (i, 0)),
            pl.BlockSpec((128, 128), lambda i: (0, 0)),
            pl.BlockSpec((128, 128), lambda i: (0, 0)),
        ],
        out_specs=[
            pl.BlockSpec((B, 128), lambda i: (i, 0)),
            pl.BlockSpec((B, 128), lambda i: (i, 0)),
            pl.BlockSpec((128, B), lambda i: (0, i)),
        ],
        out_shape=[
            jax.ShapeDtypeStruct((N, 128), jnp.float32),
            jax.ShapeDtypeStruct((N, 128), jnp.float32),
            jax.ShapeDtypeStruct((128, N), jnp.float32),
        ],
    )(X, W, M)


_LOG2E = 1.4426950408889634


def _head_mix_matrix(a_src, a_dst):
    H = a_src.shape[0]
    Dh = 128 // H
    c = jnp.arange(128)
    M = jnp.zeros((128, 128), jnp.float32)
    M = M.at[c, c // Dh].set(a_src.reshape(-1) * _LOG2E)
    M = M.at[c, H + c // Dh].set(a_dst.reshape(-1) * _LOG2E)
    return M


def _head_max(aa, H):
    mx = jnp.max(aa[:, :H], axis=0)
    return jnp.repeat(mx, 128 // H)[None, :]


def _gat_body(a_ref, xp_ref, aa_ref, aat_ref, mx_ref, b_ref, o_ref, ot_ref,
              acc, s_scr, *, H, BI, BJ, nJ):
    j = pl.program_id(1)
    Dh = 128 // H

    @pl.when(j == 0)
    def _():
        acc[...] = jnp.zeros_like(acc)
        s_scr[...] = jnp.zeros_like(s_scr)

    mask = a_ref[...] != 0

    for h in range(H):
        sl = slice(h * Dh, (h + 1) * Dh)
        adst = aa_ref[:, H + h:H + h + 1]
        asrc = aat_ref[h:h + 1, :]
        mz = adst + mx_ref[:, h * Dh:h * Dh + 1]
        m_b = jnp.maximum(mz, 0.2 * mz)
        e = adst + asrc
        e = jnp.maximum(e, 0.2 * e)
        e = jnp.where(mask, e, -1e30)
        p = jnp.exp2(e - m_b)
        s_old = s_scr[:, h * Dh:h * Dh + 1]
        s_new = s_old + jnp.sum(p, axis=1, keepdims=True)
        acc[:, sl] = acc[:, sl] + jnp.dot(
            p, xp_ref[:, sl], preferred_element_type=jnp.float32)
        s_scr[:, sl] = jnp.broadcast_to(s_new, (BI, Dh))

    @pl.when(j == nJ - 1)
    def _():
        out = jnp.maximum(acc[...] / s_scr[...] + b_ref[...], 0.0)
        o_ref[...] = out
        ot_ref[...] = out.T


def _gat_pallas(A, xp, aa, aaT, mx, b, H):
    N = A.shape[0]
    BI = min(512, N)
    BJ = min(2048, N)
    nJ = N // BJ
    body = functools.partial(_gat_body, H=H, BI=BI, BJ=BJ, nJ=nJ)
    return pl.pallas_call(
        body,
        grid=(N // BI, nJ),
        in_specs=[
            pl.BlockSpec((BI, BJ), lambda i, j: (i, j)),
            pl.BlockSpec((BJ, 128), lambda i, j: (j, 0)),
            pl.BlockSpec((BI, 128), lambda i, j: (i, 0)),
            pl.BlockSpec((128, BJ), lambda i, j: (0, j)),
            pl.BlockSpec((1, 128), lambda i, j: (0, 0)),
            pl.BlockSpec((1, 128), lambda i, j: (0, 0)),
        ],
        out_specs=[
            pl.BlockSpec((BI, 128), lambda i, j: (i, 0)),
            pl.BlockSpec((128, BI), lambda i, j: (0, i)),
        ],
        out_shape=[
            jax.ShapeDtypeStruct((N, 128), jnp.float32),
            jax.ShapeDtypeStruct((128, N), jnp.float32),
        ],
        scratch_shapes=[
            pltpu.VMEM((BI, 128), jnp.float32),
            pltpu.VMEM((BI, 128), jnp.float32),
        ],
        compiler_params=pltpu.CompilerParams(
            dimension_semantics=("parallel", "arbitrary")),
    )(A, xp, aa, aaT, mx, b.reshape(1, 128))


def _symprod_body(y_ref, yt_ref, o_ref):
    o_ref[...] = jnp.maximum(
        jnp.dot(y_ref[...], yt_ref[...], preferred_element_type=jnp.float32),
        0.0)


def _symprod(Y, Yt):
    N = Y.shape[0]
    B = 2048 if N >= 2048 else 1024
    return pl.pallas_call(
        _symprod_body,
        grid=(N // B, N // B),
        in_specs=[
            pl.BlockSpec((B, 128), lambda i, j: (i, 0)),
            pl.BlockSpec((128, B), lambda i, j: (0, j)),
        ],
        out_specs=pl.BlockSpec((B, B), lambda i, j: (i, j)),
        out_shape=jax.ShapeDtypeStruct((N, N), jnp.float32),
        compiler_params=pltpu.CompilerParams(
            dimension_semantics=("parallel", "parallel")),
    )(Y, Yt)


def _ups_body(uw_ref, x_ref, ub_ref, o_ref, ot_ref):
    xu = jnp.dot(
        uw_ref[...], x_ref[...], preferred_element_type=jnp.float32
    ) + ub_ref[...]
    o_ref[...] = xu
    ot_ref[...] = xu.T


def _upsample(uW, X, uB):
    M, N = uW.shape
    B = 1024
    return pl.pallas_call(
        _ups_body,
        grid=(M // B,),
        in_specs=[
            pl.BlockSpec((B, N), lambda i: (i, 0)),
            pl.BlockSpec((N, 128), lambda i: (0, 0)),
            pl.BlockSpec((B, 1), lambda i: (i, 0)),
        ],
        out_specs=[
            pl.BlockSpec((B, 128), lambda i: (i, 0)),
            pl.BlockSpec((128, B), lambda i: (0, i)),
        ],
        out_shape=[
            jax.ShapeDtypeStruct((M, 128), jnp.float32),
            jax.ShapeDtypeStruct((128, M), jnp.float32),
        ],
    )(uW, X, uB.reshape(M, 1))


def _expand_body(inv_ref, xs_ref, w_ref, m_ref, xp_ref, aa_ref, aat_ref,
                 *, B, K):
    t = jnp.dot(xs_ref[...], w_ref[...], preferred_element_type=jnp.float32)
    oh = (inv_ref[...] == lax.broadcasted_iota(
        jnp.int32, (B, K), 1)).astype(jnp.float32)
    xp = jnp.dot(oh, t, preferred_element_type=jnp.float32)
    xp_ref[...] = xp
    aa = jnp.dot(xp, m_ref[...], preferred_element_type=jnp.float32)
    aa_ref[...] = aa
    aat_ref[...] = aa.T


def _unpool_prep(X_small, idx, N, W, M):
    K = X_small.shape[0]
    inv = jnp.full((N, 1), K, jnp.int32).at[idx, 0].set(
        jnp.arange(K, dtype=jnp.int32))
    B = 1024
    body = functools.partial(_expand_body, B=B, K=K)
    return pl.pallas_call(
        body,
        grid=(N // B,),
        in_specs=[
            pl.BlockSpec((B, 1), lambda i: (i, 0)),
            pl.BlockSpec((K, 128), lambda i: (0, 0)),
            pl.BlockSpec((128, 128), lambda i: (0, 0)),
            pl.BlockSpec((128, 128), lambda i: (0, 0)),
        ],
        out_specs=[
            pl.BlockSpec((B, 128), lambda i: (i, 0)),
            pl.BlockSpec((B, 128), lambda i: (i, 0)),
            pl.BlockSpec((128, B), lambda i: (0, i)),
        ],
        out_shape=[
            jax.ShapeDtypeStruct((N, 128), jnp.float32),
            jax.ShapeDtypeStruct((N, 128), jnp.float32),
            jax.ShapeDtypeStruct((128, N), jnp.float32),
        ],
    )(inv, X_small, W, M)


def kernel(A, X, dW, dAs, dAd, dB, gW, gAs, gAd, gB, bW, bAs, bAd, bB,
           pW, pB, uW, uB):
    N = A.shape[0]

    An = A + jnp.eye(N, dtype=A.dtype)
    An = _sym_norm_exact(An)
    X1 = jax.nn.relu(_gat_exact(An, X, dW[0], dAs[0], dAd[0], dB[0], 4))
    w1 = (X1 @ pW[0] + pB[0])[:, 0]
    s1 = jax.nn.sigmoid(w1)
    k1 = max(2, int(0.5 * N))
    vals1, idx1 = lax.top_k(s1, k1)
    X2 = X1[idx1] * vals1[:, None]
    A2 = An[idx1][:, idx1]

    X2 = jax.nn.relu(_gat_exact(A2, X2, dW[1], dAs[1], dAd[1], dB[1], 4))
    w2 = (X2 @ pW[1] + pB[1])[:, 0]
    s2 = jax.nn.sigmoid(w2)
    k2 = max(2, int(0.5 * k1))
    vals2, idx2 = lax.top_k(s2, k2)
    X3 = X2[idx2] * vals2[:, None]
    A3 = A2[idx2][:, idx2]

    xp3, aa3, aat3 = _prep(X3, bW, _head_mix_matrix(bAs, bAd))
    Xb, _ = _gat_pallas(A3, xp3, aa3, aat3, _head_max(aa3, 2), bB, 2)

    xp4, aa4, aat4 = _unpool_prep(Xb, idx2, k1, gW[0],
                                  _head_mix_matrix(gAs[0], gAd[0]))
    X4, X4t = _gat_pallas(A2, xp4, aa4, aat4, _head_max(aa4, 4), gB[0], 4)
    A_rec0 = _symprod(X4, X4t)

    xp5, aa5, aat5 = _unpool_prep(X4, idx1, N, gW[1],
                                  _head_mix_matrix(gAs[1], gAd[1]))
    X5, X5t = _gat_pallas(An, xp5, aa5, aat5, _head_max(aa5, 4), gB[1], 4)
    A_rec1 = _symprod(X5, X5t)

    Xu, Xut = _upsample(uW, X5, uB)
    A_up = _symprod(Xu, Xut)

    return (A_up, An, A2, A_rec0, A_rec1)


# speedup vs baseline: 1.0114x; 1.0114x over previous
import functools

import jax
import jax.numpy as jnp
from jax import lax
from jax.experimental import pallas as pl
from jax.experimental.pallas import tpu as pltpu


def _gat_exact(A, X, W, a_src, a_dst, b, heads):
    N = X.shape[0]
    xp = (X @ W).reshape(N, heads, -1)
    asrc = jnp.sum(xp * a_src[None, :, :], axis=-1)
    adst = jnp.sum(xp * a_dst[None, :, :], axis=-1)
    e = adst[:, None, :] + asrc[None, :, :]
    e = jnp.where(e >= 0, e, 0.2 * e)
    mask = (A != 0) | jnp.eye(N, dtype=bool)
    e = jnp.where(mask[:, :, None], e, -1e30)
    attn = jax.nn.softmax(e, axis=1)
    out = jnp.einsum('ijh,jhd->ihd', attn, xp).reshape(N, -1)
    return out + b


def _sym_norm_exact(A):
    d = A.sum(axis=1) + 1e-5
    dinv = d ** -0.5
    return dinv[:, None] * A * dinv[None, :]


def _prep_body(x_ref, w_ref, m_ref, xp_ref, aa_ref, aat_ref):
    xp = jnp.dot(x_ref[...], w_ref[...], preferred_element_type=jnp.float32)
    xp_ref[...] = xp
    aa = jnp.dot(xp, m_ref[...], preferred_element_type=jnp.float32)
    aa_ref[...] = aa
    aat_ref[...] = aa.T


def _prep(X, W, M):
    N = X.shape[0]
    B = min(1024, N)
    return pl.pallas_call(
        _prep_body,
        grid=(N // B,),
        in_specs=[
            pl.BlockSpec((B, 128), lambda i: (i, 0)),
            pl.BlockSpec((128, 128), lambda i: (0, 0)),
            pl.BlockSpec((128, 128), lambda i: (0, 0)),
        ],
        out_specs=[
            pl.BlockSpec((B, 128), lambda i: (i, 0)),
            pl.BlockSpec((B, 128), lambda i: (i, 0)),
            pl.BlockSpec((128, B), lambda i: (0, i)),
        ],
        out_shape=[
            jax.ShapeDtypeStruct((N, 128), jnp.float32),
            jax.ShapeDtypeStruct((N, 128), jnp.float32),
            jax.ShapeDtypeStruct((128, N), jnp.float32),
        ],
    )(X, W, M)


_LOG2E = 1.4426950408889634


def _head_mix_matrix(a_src, a_dst):
    H = a_src.shape[0]
    Dh = 128 // H
    c = jnp.arange(128)
    M = jnp.zeros((128, 128), jnp.float32)
    M = M.at[c, c // Dh].set(a_src.reshape(-1) * _LOG2E)
    M = M.at[c, H + c // Dh].set(a_dst.reshape(-1) * _LOG2E)
    return M


def _head_max(aa, H):
    mx = jnp.max(aa[:, :H], axis=0)
    return jnp.repeat(mx, 128 // H)[None, :]


def _gat_body(a_ref, xp_ref, aa_ref, aat_ref, mx_ref, b_ref, o_ref, ot_ref,
              acc, s_scr, *, H, BI, BJ, nJ):
    j = pl.program_id(1)
    Dh = 128 // H

    @pl.when(j == 0)
    def _():
        acc[...] = jnp.zeros_like(acc)
        s_scr[...] = jnp.zeros_like(s_scr)

    mask = a_ref[...] != 0

    for h in range(H):
        sl = slice(h * Dh, (h + 1) * Dh)
        adst = aa_ref[:, H + h:H + h + 1]
        asrc = aat_ref[h:h + 1, :]
        mz = adst + mx_ref[:, h * Dh:h * Dh + 1]
        m_b = jnp.maximum(mz, 0.2 * mz)
        e = adst + asrc
        e = jnp.maximum(e, 0.2 * e)
        e = jnp.where(mask, e, -1e30)
        p = jnp.exp2(e - m_b)
        s_old = s_scr[:, h * Dh:h * Dh + 1]
        s_new = s_old + jnp.sum(p, axis=1, keepdims=True)
        acc[:, sl] = acc[:, sl] + jnp.dot(
            p, xp_ref[:, sl], preferred_element_type=jnp.float32)
        s_scr[:, sl] = jnp.broadcast_to(s_new, (BI, Dh))

    @pl.when(j == nJ - 1)
    def _():
        out = jnp.maximum(acc[...] / s_scr[...] + b_ref[...], 0.0)
        o_ref[...] = out
        ot_ref[...] = out.T


def _gat_pallas(A, xp, aa, aaT, mx, b, H):
    N = A.shape[0]
    BI = min(1024, N)
    BJ = min(2048, N)
    nJ = N // BJ
    body = functools.partial(_gat_body, H=H, BI=BI, BJ=BJ, nJ=nJ)
    return pl.pallas_call(
        body,
        grid=(N // BI, nJ),
        in_specs=[
            pl.BlockSpec((BI, BJ), lambda i, j: (i, j)),
            pl.BlockSpec((BJ, 128), lambda i, j: (j, 0)),
            pl.BlockSpec((BI, 128), lambda i, j: (i, 0)),
            pl.BlockSpec((128, BJ), lambda i, j: (0, j)),
            pl.BlockSpec((1, 128), lambda i, j: (0, 0)),
            pl.BlockSpec((1, 128), lambda i, j: (0, 0)),
        ],
        out_specs=[
            pl.BlockSpec((BI, 128), lambda i, j: (i, 0)),
            pl.BlockSpec((128, BI), lambda i, j: (0, i)),
        ],
        out_shape=[
            jax.ShapeDtypeStruct((N, 128), jnp.float32),
            jax.ShapeDtypeStruct((128, N), jnp.float32),
        ],
        scratch_shapes=[
            pltpu.VMEM((BI, 128), jnp.float32),
            pltpu.VMEM((BI, 128), jnp.float32),
        ],
        compiler_params=pltpu.CompilerParams(
            dimension_semantics=("parallel", "arbitrary")),
    )(A, xp, aa, aaT, mx, b.reshape(1, 128))


def _symprod_body(y_ref, yt_ref, o_ref):
    o_ref[...] = jnp.maximum(
        jnp.dot(y_ref[...], yt_ref[...], preferred_element_type=jnp.float32),
        0.0)


def _symprod(Y, Yt):
    N = Y.shape[0]
    B = 2048 if N >= 2048 else 1024
    return pl.pallas_call(
        _symprod_body,
        grid=(N // B, N // B),
        in_specs=[
            pl.BlockSpec((B, 128), lambda i, j: (i, 0)),
            pl.BlockSpec((128, B), lambda i, j: (0, j)),
        ],
        out_specs=pl.BlockSpec((B, B), lambda i, j: (i, j)),
        out_shape=jax.ShapeDtypeStruct((N, N), jnp.float32),
        compiler_params=pltpu.CompilerParams(
            dimension_semantics=("parallel", "parallel")),
    )(Y, Yt)


def _ups_body(uw_ref, x_ref, ub_ref, o_ref, ot_ref):
    xu = jnp.dot(
        uw_ref[...], x_ref[...], preferred_element_type=jnp.float32
    ) + ub_ref[...]
    o_ref[...] = xu
    ot_ref[...] = xu.T


def _upsample(uW, X, uB):
    M, N = uW.shape
    B = 1024
    return pl.pallas_call(
        _ups_body,
        grid=(M // B,),
        in_specs=[
            pl.BlockSpec((B, N), lambda i: (i, 0)),
            pl.BlockSpec((N, 128), lambda i: (0, 0)),
            pl.BlockSpec((B, 1), lambda i: (i, 0)),
        ],
        out_specs=[
            pl.BlockSpec((B, 128), lambda i: (i, 0)),
            pl.BlockSpec((128, B), lambda i: (0, i)),
        ],
        out_shape=[
            jax.ShapeDtypeStruct((M, 128), jnp.float32),
            jax.ShapeDtypeStruct((128, M), jnp.float32),
        ],
    )(uW, X, uB.reshape(M, 1))


def _expand_body(inv_ref, xs_ref, w_ref, m_ref, xp_ref, aa_ref, aat_ref,
                 *, B, K):
    t = jnp.dot(xs_ref[...], w_ref[...], preferred_element_type=jnp.float32)
    oh = (inv_ref[...] == lax.broadcasted_iota(
        jnp.int32, (B, K), 1)).astype(jnp.float32)
    xp = jnp.dot(oh, t, preferred_element_type=jnp.float32)
    xp_ref[...] = xp
    aa = jnp.dot(xp, m_ref[...], preferred_element_type=jnp.float32)
    aa_ref[...] = aa
    aat_ref[...] = aa.T


def _unpool_prep(X_small, idx, N, W, M):
    K = X_small.shape[0]
    inv = jnp.full((N, 1), K, jnp.int32).at[idx, 0].set(
        jnp.arange(K, dtype=jnp.int32))
    B = 1024
    body = functools.partial(_expand_body, B=B, K=K)
    return pl.pallas_call(
        body,
        grid=(N // B,),
        in_specs=[
            pl.BlockSpec((B, 1), lambda i: (i, 0)),
            pl.BlockSpec((K, 128), lambda i: (0, 0)),
            pl.BlockSpec((128, 128), lambda i: (0, 0)),
            pl.BlockSpec((128, 128), lambda i: (0, 0)),
        ],
        out_specs=[
            pl.BlockSpec((B, 128), lambda i: (i, 0)),
            pl.BlockSpec((B, 128), lambda i: (i, 0)),
            pl.BlockSpec((128, B), lambda i: (0, i)),
        ],
        out_shape=[
            jax.ShapeDtypeStruct((N, 128), jnp.float32),
            jax.ShapeDtypeStruct((N, 128), jnp.float32),
            jax.ShapeDtypeStruct((128, N), jnp.float32),
        ],
    )(inv, X_small, W, M)


def kernel(A, X, dW, dAs, dAd, dB, gW, gAs, gAd, gB, bW, bAs, bAd, bB,
           pW, pB, uW, uB):
    N = A.shape[0]

    An = A + jnp.eye(N, dtype=A.dtype)
    An = _sym_norm_exact(An)
    X1 = jax.nn.relu(_gat_exact(An, X, dW[0], dAs[0], dAd[0], dB[0], 4))
    w1 = (X1 @ pW[0] + pB[0])[:, 0]
    s1 = jax.nn.sigmoid(w1)
    k1 = max(2, int(0.5 * N))
    vals1, idx1 = lax.top_k(s1, k1)
    X2 = X1[idx1] * vals1[:, None]
    A2 = An[idx1][:, idx1]

    X2 = jax.nn.relu(_gat_exact(A2, X2, dW[1], dAs[1], dAd[1], dB[1], 4))
    w2 = (X2 @ pW[1] + pB[1])[:, 0]
    s2 = jax.nn.sigmoid(w2)
    k2 = max(2, int(0.5 * k1))
    vals2, idx2 = lax.top_k(s2, k2)
    X3 = X2[idx2] * vals2[:, None]
    A3 = A2[idx2][:, idx2]

    xp3, aa3, aat3 = _prep(X3, bW, _head_mix_matrix(bAs, bAd))
    Xb, _ = _gat_pallas(A3, xp3, aa3, aat3, _head_max(aa3, 2), bB, 2)

    xp4, aa4, aat4 = _unpool_prep(Xb, idx2, k1, gW[0],
                                  _head_mix_matrix(gAs[0], gAd[0]))
    X4, X4t = _gat_pallas(A2, xp4, aa4, aat4, _head_max(aa4, 4), gB[0], 4)
    A_rec0 = _symprod(X4, X4t)

    xp5, aa5, aat5 = _unpool_prep(X4, idx1, N, gW[1],
                                  _head_mix_matrix(gAs[1], gAd[1]))
    X5, X5t = _gat_pallas(An, xp5, aa5, aat5, _head_max(aa5, 4), gB[1], 4)
    A_rec1 = _symprod(X5, X5t)

    Xu, Xut = _upsample(uW, X5, uB)
    A_up = _symprod(Xu, Xut)

    return (A_up, An, A2, A_rec0, A_rec1)
